# Initial kernel scaffold; baseline (speedup 1.0000x reference)
import jax
import jax.numpy as jnp
from jax.experimental import pallas as pl


def kernel(target_x, query_x, target_edge_index, query_edge_index, mask, embed, Wl, bl, Wr, Wsim, coef_t, coef_q):
    raise NotImplementedError("write your pallas kernel here")


# speedup vs baseline: 5.4782x; 5.4782x over previous
import functools
import math

import jax
import jax.numpy as jnp
from jax import lax
from jax.experimental import pallas as pl
from jax.experimental.pallas import tpu as pltpu
from jax.experimental.pallas import tpu_sc as plsc

NT = 10000
NQ = 256
ET = 320000
EQ = 1024
H = 128
L = 3
NTP = 10240
BT = 2048
NBT = NTP // BT
NC = 2
NS = 16
NW = NC * NS
EPW = ET // NW
KC = 80
NCHUNK = EPW // KC
ROWS_PT = NTP // NS
TPW = NTP // NW
QPW = NQ // NW
CW = 128

_SC_MESH = dict(core_axis_name="c", subcore_axis_name="s")


def _gather_cnt_body(embed_hbm, txp_hbm, qx_hbm, dst2d_hbm, zc_hbm, ones_hbm,
                     et0_hbm, eq0_hbm, cnt_hbm,
                     cnt_sh, idx80, qidx_v, dstc_v, rows_v, ones_v, sem):
    c = lax.axis_index("c")
    s = lax.axis_index("s")
    wid = c * NS + s
    my_rows = pl.multiple_of(s * ROWS_PT, 8)
    pltpu.sync_copy(zc_hbm, cnt_sh.at[pl.ds(my_rows, ROWS_PT)])
    pltpu.sync_copy(ones_hbm, ones_v)

    tb = pl.multiple_of(wid * TPW, 8)

    def tchunk(j, carry):
        off = pl.multiple_of(j * KC, 8)
        pltpu.sync_copy(txp_hbm.at[pl.ds(tb + off, KC)], idx80)
        pltpu.async_copy(embed_hbm.at[idx80], rows_v, sem).wait()
        pltpu.sync_copy(rows_v, et0_hbm.at[pl.ds(tb + off, KC)])
        return carry

    lax.fori_loop(0, TPW // KC, tchunk, 0)

    qb = pl.multiple_of(wid * QPW, 8)
    pltpu.sync_copy(qx_hbm.at[pl.ds(qb, QPW)], qidx_v)
    pltpu.async_copy(embed_hbm.at[qidx_v], rows_v.at[pl.ds(0, QPW)], sem).wait()
    pltpu.sync_copy(rows_v.at[pl.ds(0, QPW)], eq0_hbm.at[pl.ds(qb, QPW)])

    pltpu.sync_copy(dst2d_hbm.at[wid], dstc_v)
    plsc.subcore_barrier()

    def cchunk(j, carry):
        pltpu.sync_copy(ones_v, cnt_sh.at[dstc_v.at[j]], add=True)
        return carry

    lax.fori_loop(0, NCHUNK, cchunk, 0)
    plsc.subcore_barrier()
    pltpu.sync_copy(cnt_sh.at[pl.ds(my_rows, ROWS_PT)],
                    cnt_hbm.at[pl.ds(c * NTP + my_rows, ROWS_PT)])


def _gather_cnt(embed, txp, qx, dst2d, zc, ones_c):
    fn = pl.kernel(
        _gather_cnt_body,
        out_type=[
            jax.ShapeDtypeStruct((NTP, H), jnp.float32),
            jax.ShapeDtypeStruct((NQ, H), jnp.float32),
            jax.ShapeDtypeStruct((NC * NTP, CW), jnp.float32),
        ],
        mesh=plsc.VectorSubcoreMesh(**_SC_MESH),
        scratch_types=[
            pltpu.VMEM_SHARED((NTP, CW), jnp.float32),
            pltpu.VMEM((KC,), jnp.int32),
            pltpu.VMEM((QPW,), jnp.int32),
            pltpu.VMEM((NCHUNK, KC), jnp.int32),
            pltpu.VMEM((KC, H), jnp.float32),
            pltpu.VMEM((KC, CW), jnp.float32),
            pltpu.SemaphoreType.DMA,
        ],
    )
    return fn(embed, txp, qx, dst2d, zc, ones_c)


def _seg_sum_body(x_hbm, src_hbm, dst2d_hbm, za_hbm, out_hbm,
                  agg_sh, src_v, dstc_v, rows_v, sem):
    c = lax.axis_index("c")
    s = lax.axis_index("s")
    wid = c * NS + s
    my_rows = pl.multiple_of(s * ROWS_PT, 8)
    ebase = pl.multiple_of(wid * EPW, 8)
    pltpu.sync_copy(za_hbm, agg_sh.at[pl.ds(my_rows, ROWS_PT)])
    pltpu.sync_copy(src_hbm.at[pl.ds(ebase, EPW)], src_v)
    pltpu.sync_copy(dst2d_hbm.at[wid], dstc_v)
    plsc.subcore_barrier()

    def chunk(j, carry):
        off = pl.multiple_of(j * KC, 8)
        pltpu.async_copy(x_hbm.at[src_v.at[pl.ds(off, KC)]], rows_v, sem).wait()
        pltpu.sync_copy(rows_v, agg_sh.at[dstc_v.at[j]], add=True)
        return carry

    lax.fori_loop(0, NCHUNK, chunk, 0)
    plsc.subcore_barrier()
    pltpu.sync_copy(agg_sh.at[pl.ds(my_rows, ROWS_PT)],
                    out_hbm.at[pl.ds(c * NTP + my_rows, ROWS_PT)])


def _seg_sum(x, src, dst2d, za):
    fn = pl.kernel(
        _seg_sum_body,
        out_type=jax.ShapeDtypeStruct((NC * NTP, H), jnp.float32),
        mesh=plsc.VectorSubcoreMesh(**_SC_MESH),
        scratch_types=[
            pltpu.VMEM_SHARED((NTP, H), jnp.float32),
            pltpu.VMEM((EPW,), jnp.int32),
            pltpu.VMEM((NCHUNK, KC), jnp.int32),
            pltpu.VMEM((KC, H), jnp.float32),
            pltpu.SemaphoreType.DMA,
        ],
    )
    return fn(x, src, dst2d, za)


def _prep_body(qei_ref, mq_ref):
    src = qei_ref[0:1, :]
    dst = qei_ref[1:2, :]
    ids = lax.broadcasted_iota(jnp.int32, (NQ, EQ), 0)
    ohd = (ids == dst).astype(jnp.float32)
    ohs = (ids == src).astype(jnp.float32)
    aq = lax.dot_general(ohd, ohs, (((1,), (1,)), ((), ())),
                         preferred_element_type=jnp.float32)
    cnt = jnp.sum(aq, axis=1, keepdims=True)
    mq_ref[...] = aq / jnp.maximum(cnt, 1.0)


def _prep(qei):
    return pl.pallas_call(
        _prep_body,
        out_shape=jax.ShapeDtypeStruct((NQ, NQ), jnp.float32),
    )(qei)


def _elu(x):
    return jnp.where(x > 0, x, jnp.exp(jnp.minimum(x, 0.0)) - 1.0)


def _mmt(a, b):
    return lax.dot_general(a, b, (((1,), (1,)), ((), ())),
                           preferred_element_type=jnp.float32)


def _sage_body(xt_ref, agg_ref, cnt_ref, wl_ref, bl_ref, wr_ref,
               xq_ref, mq_ref, ht_ref, hq_ref):
    j = pl.program_id(0)
    agg = agg_ref[0] + agg_ref[1]
    cnt = cnt_ref[0, :, 0:1] + cnt_ref[1, :, 0:1]
    mean = agg / jnp.maximum(cnt, 1.0)
    h = _mmt(mean, wl_ref[...]) + bl_ref[...] + _mmt(xt_ref[...], wr_ref[...])
    ht_ref[...] = _elu(h)

    @pl.when(j == 0)
    def _():
        meanq = jnp.dot(mq_ref[...], xq_ref[...],
                        preferred_element_type=jnp.float32)
        hq = _mmt(meanq, wl_ref[...]) + bl_ref[...] + _mmt(xq_ref[...],
                                                           wr_ref[...])
        hq_ref[...] = _elu(hq)


def _sage(xt, agg2, cnt2, wl, bl, wr, xq, mq):
    return pl.pallas_call(
        _sage_body,
        grid=(NBT,),
        in_specs=[
            pl.BlockSpec((BT, H), lambda j: (j, 0)),
            pl.BlockSpec((NC, BT, H), lambda j: (0, j, 0)),
            pl.BlockSpec((NC, BT, CW), lambda j: (0, j, 0)),
            pl.BlockSpec((H, H), lambda j: (0, 0)),
            pl.BlockSpec((1, H), lambda j: (0, 0)),
            pl.BlockSpec((H, H), lambda j: (0, 0)),
            pl.BlockSpec((NQ, H), lambda j: (0, 0)),
            pl.BlockSpec((NQ, NQ), lambda j: (0, 0)),
        ],
        out_specs=[
            pl.BlockSpec((BT, H), lambda j: (j, 0)),
            pl.BlockSpec((NQ, H), lambda j: (0, 0)),
        ],
        out_shape=[
            jax.ShapeDtypeStruct((NTP, H), jnp.float32),
            jax.ShapeDtypeStruct((NQ, H), jnp.float32),
        ],
    )(xt, agg2, cnt2, wl, bl, wr, xq, mq)


def _att_mix_body(ht_ref, hq_ref, et0_ref, eq0_ref, wsim_ref, ct_ref, cq_ref,
                  att_ref, xt_ref, xq_ref):
    ht = ht_ref[...]
    hq = hq_ref[...]
    sq = jnp.dot(hq, wsim_ref[...], preferred_element_type=jnp.float32)
    logits = _mmt(sq, ht) * (1.0 / math.sqrt(H))
    col = lax.broadcasted_iota(jnp.int32, (NQ, NTP), 1)
    logits = jnp.where(col < NT, logits, -1e9)
    m = jnp.max(logits, axis=1, keepdims=True)
    e = jnp.exp(logits - m)
    att = e / jnp.sum(e, axis=1, keepdims=True)
    att_ref[...] = att
    cross_q = jnp.dot(att, ht, preferred_element_type=jnp.float32)
    cross_t = lax.dot_general(att, hq, (((0,), (0,)), ((), ())),
                              preferred_element_type=jnp.float32)
    ct = ct_ref[...]
    cq = cq_ref[...]
    xq_ref[...] = (cq[0:1, 0:1] * eq0_ref[...] + cq[0:1, 1:2] * hq
                   + cq[0:1, 2:3] * cross_q)
    xt_ref[...] = (ct[0:1, 0:1] * et0_ref[...] + ct[0:1, 1:2] * ht
                   + ct[0:1, 2:3] * cross_t)


def _att_mix(ht, hq, et0, eq0, wsim, ct, cq):
    return pl.pallas_call(
        _att_mix_body,
        out_shape=[
            jax.ShapeDtypeStruct((NQ, NTP), jnp.float32),
            jax.ShapeDtypeStruct((NTP, H), jnp.float32),
            jax.ShapeDtypeStruct((NQ, H), jnp.float32),
        ],
    )(ht, hq, et0, eq0, wsim, ct, cq)


def _final_att_body(xt_ref, xq_ref, att_ref):
    logits = _mmt(xq_ref[...], xt_ref[...]) * (1.0 / math.sqrt(H))
    col = lax.broadcasted_iota(jnp.int32, (NQ, NTP), 1)
    logits = jnp.where(col < NT, logits, -1e9)
    m = jnp.max(logits, axis=1, keepdims=True)
    e = jnp.exp(logits - m)
    att_ref[...] = e / jnp.sum(e, axis=1, keepdims=True)


def _final_att(xt, xq):
    return pl.pallas_call(
        _final_att_body,
        out_shape=jax.ShapeDtypeStruct((NQ, NTP), jnp.float32),
    )(xt, xq)


def kernel(target_x, query_x, target_edge_index, query_edge_index, mask,
           embed, Wl, bl, Wr, Wsim, coef_t, coef_q):
    del mask
    tx = target_x.astype(jnp.int32)
    qx = query_x.astype(jnp.int32)
    txp = jnp.pad(tx, (0, NTP - NT))
    src = target_edge_index[0].astype(jnp.int32)
    dst2d = target_edge_index[1].astype(jnp.int32).reshape(NW, NCHUNK, KC)
    zc = jnp.zeros((ROWS_PT, CW), jnp.float32)
    za = jnp.zeros((ROWS_PT, H), jnp.float32)
    ones_c = jnp.ones((KC, CW), jnp.float32)

    embed_p = jnp.pad(embed, ((0, -embed.shape[0] % 8), (0, 0)))
    et0, eq0, cnt2f = _gather_cnt(embed_p, txp, qx, dst2d, zc, ones_c)
    cnt2 = cnt2f.reshape(NC, NTP, CW)
    mq = _prep(query_edge_index.astype(jnp.int32))

    xt, xq = et0, eq0
    atts = []
    for i in range(L):
        agg2 = _seg_sum(xt, src, dst2d, za).reshape(NC, NTP, H)
        ht, hq = _sage(xt, agg2, cnt2, Wl[i], bl[i].reshape(1, H), Wr[i],
                       xq, mq)
        att, xt, xq = _att_mix(ht, hq, et0, eq0, Wsim[i],
                               coef_t[i].reshape(1, 3),
                               coef_q[i].reshape(1, 3))
        atts.append(att)
    atts.append(_final_att(xt, xq))
    return jnp.stack(atts)[:, :, :NT]


# speedup vs baseline: 6.5895x; 1.2029x over previous
import functools
import math

import jax
import jax.numpy as jnp
from jax import lax
from jax.experimental import pallas as pl
from jax.experimental.pallas import tpu as pltpu
from jax.experimental.pallas import tpu_sc as plsc

NT = 10000
NQ = 256
ET = 320000
EQ = 1024
H = 128
L = 3
NTP = 10240
BT = 2048
NBT = NTP // BT
NC = 2
NS = 16
NW = NC * NS
EPW = ET // NW
KC = 80
NCHUNK = EPW // KC
ROWS_PT = NTP // NS
TPW = NTP // NW
QPW = NQ // NW

_SC_MESH = dict(core_axis_name="c", subcore_axis_name="s")


def _gather_cnt_body(embed_hbm, txp_hbm, qx_hbm, dst2d_hbm, zc_hbm, ones_hbm,
                     et0_hbm, eq0_hbm, cnt_hbm,
                     cnt_sh, idx80, qidx_v, dstc_v, rows_v, ones_v, sem):
    c = lax.axis_index("c")
    s = lax.axis_index("s")
    wid = c * NS + s
    my_rows = pl.multiple_of(s * ROWS_PT, 8)
    pltpu.sync_copy(zc_hbm, cnt_sh.at[pl.ds(my_rows, ROWS_PT)])
    pltpu.sync_copy(ones_hbm, ones_v)

    tb = pl.multiple_of(wid * TPW, 8)

    def tchunk(j, carry):
        off = pl.multiple_of(j * KC, 8)
        pltpu.sync_copy(txp_hbm.at[pl.ds(tb + off, KC)], idx80)
        pltpu.async_copy(embed_hbm.at[idx80], rows_v, sem).wait()
        pltpu.sync_copy(rows_v, et0_hbm.at[pl.ds(tb + off, KC)])
        return carry

    lax.fori_loop(0, TPW // KC, tchunk, 0)

    qb = pl.multiple_of(wid * QPW, 8)
    pltpu.sync_copy(qx_hbm.at[pl.ds(qb, QPW)], qidx_v)
    pltpu.async_copy(embed_hbm.at[qidx_v], rows_v.at[pl.ds(0, QPW)], sem).wait()
    pltpu.sync_copy(rows_v.at[pl.ds(0, QPW)], eq0_hbm.at[pl.ds(qb, QPW)])

    pltpu.sync_copy(dst2d_hbm.at[wid], dstc_v)
    plsc.subcore_barrier()

    def cchunk(j, carry):
        pltpu.sync_copy(ones_v, cnt_sh.at[dstc_v.at[j]], add=True)
        return carry

    lax.fori_loop(0, NCHUNK, cchunk, 0)
    plsc.subcore_barrier()
    pltpu.sync_copy(cnt_sh.at[pl.ds(my_rows, ROWS_PT)],
                    cnt_hbm.at[pl.ds(c * NTP + my_rows, ROWS_PT)])


def _gather_cnt(embed, txp, qx, dst2d, zc, ones_c):
    fn = pl.kernel(
        _gather_cnt_body,
        out_type=[
            jax.ShapeDtypeStruct((NTP, H), jnp.float32),
            jax.ShapeDtypeStruct((NQ, H), jnp.float32),
            jax.ShapeDtypeStruct((NC * NTP, H), jnp.float32),
        ],
        mesh=plsc.VectorSubcoreMesh(**_SC_MESH),
        scratch_types=[
            pltpu.VMEM_SHARED((NTP, H), jnp.float32),
            pltpu.VMEM((KC,), jnp.int32),
            pltpu.VMEM((QPW,), jnp.int32),
            pltpu.VMEM((NCHUNK, KC), jnp.int32),
            pltpu.VMEM((KC, H), jnp.float32),
            pltpu.VMEM((KC, H), jnp.float32),
            pltpu.SemaphoreType.DMA,
        ],
    )
    return fn(embed, txp, qx, dst2d, zc, ones_c)


NBUF = 2


def _seg_sum_body(x_hbm, src_hbm, dst2d_hbm, za_hbm, out_hbm,
                  agg_sh, src_v, dstc_v, r0, r1, sg0, sg1, ss0, ss1):
    c = lax.axis_index("c")
    s = lax.axis_index("s")
    wid = c * NS + s
    my_rows = pl.multiple_of(s * ROWS_PT, 8)
    ebase = pl.multiple_of(wid * EPW, 8)
    pltpu.sync_copy(za_hbm, agg_sh.at[pl.ds(my_rows, ROWS_PT)])
    pltpu.sync_copy(src_hbm.at[pl.ds(ebase, EPW)], src_v)
    pltpu.sync_copy(dst2d_hbm.at[wid], dstc_v)

    rows = (r0, r1)
    sgs = (sg0, sg1)
    sss = (ss0, ss1)

    def g_desc(chk, b):
        off = pl.multiple_of(chk * KC, 8)
        return pltpu.make_async_copy(x_hbm.at[src_v.at[pl.ds(off, KC)]],
                                     rows[b], sgs[b])

    def s_desc(chk, b):
        return pltpu.make_async_copy(rows[b], agg_sh.at[dstc_v.at[chk]],
                                     sss[b])

    for b in range(NBUF - 1):
        g_desc(b, b).start()
    plsc.subcore_barrier()

    def quad(jj, carry):
        j0 = jj * NBUF
        for b in range(NBUF):
            chk = j0 + b
            g_desc(chk, b).wait()
            s_desc(chk, b).start(add=True)
            nxt = chk + (NBUF - 1)
            q = (b + NBUF - 1) % NBUF

            @pl.when(nxt < NCHUNK)
            def _():
                @pl.when(chk >= 1)
                def _():
                    s_desc(chk - 1, q).wait()

                g_desc(nxt, q).start()
        return carry

    lax.fori_loop(0, NCHUNK // NBUF, quad, 0)
    tail = NCHUNK - 1
    g_desc(tail, tail % NBUF).wait()
    s_desc(tail, tail % NBUF).start(add=True)
    for chk in range(NCHUNK - NBUF, NCHUNK):
        s_desc(chk, chk % NBUF).wait()
    plsc.subcore_barrier()
    pltpu.sync_copy(agg_sh.at[pl.ds(my_rows, ROWS_PT)],
                    out_hbm.at[pl.ds(c * NTP + my_rows, ROWS_PT)])


def _seg_sum(x, src, dst2d, za):
    fn = pl.kernel(
        _seg_sum_body,
        out_type=jax.ShapeDtypeStruct((NC * NTP, H), jnp.float32),
        mesh=plsc.VectorSubcoreMesh(**_SC_MESH),
        scratch_types=[
            pltpu.VMEM_SHARED((NTP, H), jnp.float32),
            pltpu.VMEM((EPW,), jnp.int32),
            pltpu.VMEM((NCHUNK, KC), jnp.int32),
        ] + [pltpu.VMEM((KC, H), jnp.float32)] * NBUF
          + [pltpu.SemaphoreType.DMA] * (2 * NBUF),
    )
    return fn(x, src, dst2d, za)


def _prep_body(qei_ref, mq_ref):
    src = qei_ref[0:1, :]
    dst = qei_ref[1:2, :]
    ids = lax.broadcasted_iota(jnp.int32, (NQ, EQ), 0)
    ohd = (ids == dst).astype(jnp.float32)
    ohs = (ids == src).astype(jnp.float32)
    aq = lax.dot_general(ohd, ohs, (((1,), (1,)), ((), ())),
                         preferred_element_type=jnp.float32)
    cnt = jnp.sum(aq, axis=1, keepdims=True)
    mq_ref[...] = aq / jnp.maximum(cnt, 1.0)


def _prep(qei):
    return pl.pallas_call(
        _prep_body,
        out_shape=jax.ShapeDtypeStruct((NQ, NQ), jnp.float32),
    )(qei)


def _elu(x):
    return jnp.where(x > 0, x, jnp.exp(jnp.minimum(x, 0.0)) - 1.0)


def _mmt(a, b):
    return lax.dot_general(a, b, (((1,), (1,)), ((), ())),
                           preferred_element_type=jnp.float32)


def _sage_body(xt_ref, agg_ref, cnt_ref, wl_ref, bl_ref, wr_ref,
               xq_ref, mq_ref, ht_ref, hq_ref):
    j = pl.program_id(0)
    agg = agg_ref[0] + agg_ref[1]
    cnt = cnt_ref[0, :, 0:1] + cnt_ref[1, :, 0:1]
    mean = agg / jnp.maximum(cnt, 1.0)
    h = _mmt(mean, wl_ref[...]) + bl_ref[...] + _mmt(xt_ref[...], wr_ref[...])
    ht_ref[...] = _elu(h)

    @pl.when(j == 0)
    def _():
        meanq = jnp.dot(mq_ref[...], xq_ref[...],
                        preferred_element_type=jnp.float32)
        hq = _mmt(meanq, wl_ref[...]) + bl_ref[...] + _mmt(xq_ref[...],
                                                           wr_ref[...])
        hq_ref[...] = _elu(hq)


def _sage(xt, agg2, cnt2, wl, bl, wr, xq, mq):
    return pl.pallas_call(
        _sage_body,
        grid=(NBT,),
        in_specs=[
            pl.BlockSpec((BT, H), lambda j: (j, 0)),
            pl.BlockSpec((NC, BT, H), lambda j: (0, j, 0)),
            pl.BlockSpec((NC, BT, H), lambda j: (0, j, 0)),
            pl.BlockSpec((H, H), lambda j: (0, 0)),
            pl.BlockSpec((1, H), lambda j: (0, 0)),
            pl.BlockSpec((H, H), lambda j: (0, 0)),
            pl.BlockSpec((NQ, H), lambda j: (0, 0)),
            pl.BlockSpec((NQ, NQ), lambda j: (0, 0)),
        ],
        out_specs=[
            pl.BlockSpec((BT, H), lambda j: (j, 0)),
            pl.BlockSpec((NQ, H), lambda j: (0, 0)),
        ],
        out_shape=[
            jax.ShapeDtypeStruct((NTP, H), jnp.float32),
            jax.ShapeDtypeStruct((NQ, H), jnp.float32),
        ],
    )(xt, agg2, cnt2, wl, bl, wr, xq, mq)


def _att_mix_body(ht_ref, hq_ref, et0_ref, eq0_ref, wsim_ref, ct_ref, cq_ref,
                  att_ref, xt_ref, xq_ref):
    ht = ht_ref[...]
    hq = hq_ref[...]
    sq = jnp.dot(hq, wsim_ref[...], preferred_element_type=jnp.float32)
    logits = _mmt(sq, ht) * (1.0 / math.sqrt(H))
    col = lax.broadcasted_iota(jnp.int32, (NQ, NTP), 1)
    logits = jnp.where(col < NT, logits, -1e9)
    m = jnp.max(logits, axis=1, keepdims=True)
    e = jnp.exp(logits - m)
    att = e / jnp.sum(e, axis=1, keepdims=True)
    att_ref[...] = att
    cross_q = jnp.dot(att, ht, preferred_element_type=jnp.float32)
    cross_t = lax.dot_general(att, hq, (((0,), (0,)), ((), ())),
                              preferred_element_type=jnp.float32)
    ct = ct_ref[...]
    cq = cq_ref[...]
    xq_ref[...] = (cq[0:1, 0:1] * eq0_ref[...] + cq[0:1, 1:2] * hq
                   + cq[0:1, 2:3] * cross_q)
    xt_ref[...] = (ct[0:1, 0:1] * et0_ref[...] + ct[0:1, 1:2] * ht
                   + ct[0:1, 2:3] * cross_t)


def _att_mix(ht, hq, et0, eq0, wsim, ct, cq):
    return pl.pallas_call(
        _att_mix_body,
        out_shape=[
            jax.ShapeDtypeStruct((NQ, NTP), jnp.float32),
            jax.ShapeDtypeStruct((NTP, H), jnp.float32),
            jax.ShapeDtypeStruct((NQ, H), jnp.float32),
        ],
    )(ht, hq, et0, eq0, wsim, ct, cq)


def _final_att_body(xt_ref, xq_ref, att_ref):
    logits = _mmt(xq_ref[...], xt_ref[...]) * (1.0 / math.sqrt(H))
    col = lax.broadcasted_iota(jnp.int32, (NQ, NTP), 1)
    logits = jnp.where(col < NT, logits, -1e9)
    m = jnp.max(logits, axis=1, keepdims=True)
    e = jnp.exp(logits - m)
    att_ref[...] = e / jnp.sum(e, axis=1, keepdims=True)


def _final_att(xt, xq):
    return pl.pallas_call(
        _final_att_body,
        out_shape=jax.ShapeDtypeStruct((NQ, NTP), jnp.float32),
    )(xt, xq)


def kernel(target_x, query_x, target_edge_index, query_edge_index, mask,
           embed, Wl, bl, Wr, Wsim, coef_t, coef_q):
    del mask
    tx = target_x.astype(jnp.int32)
    qx = query_x.astype(jnp.int32)
    txp = jnp.pad(tx, (0, NTP - NT))
    src = target_edge_index[0].astype(jnp.int32)
    dst2d = target_edge_index[1].astype(jnp.int32).reshape(NW, NCHUNK, KC)
    za = jnp.zeros((ROWS_PT, H), jnp.float32)
    ones_c = jnp.ones((KC, H), jnp.float32)

    embed_p = jnp.pad(embed, ((0, -embed.shape[0] % 8), (0, 0)))
    et0, eq0, cnt2f = _gather_cnt(embed_p, txp, qx, dst2d, za, ones_c)
    cnt2 = cnt2f.reshape(NC, NTP, H)
    mq = _prep(query_edge_index.astype(jnp.int32))

    xt, xq = et0, eq0
    atts = []
    for i in range(L):
        agg2 = _seg_sum(xt, src, dst2d, za).reshape(NC, NTP, H)
        ht, hq = _sage(xt, agg2, cnt2, Wl[i], bl[i].reshape(1, H), Wr[i],
                       xq, mq)
        att, xt, xq = _att_mix(ht, hq, et0, eq0, Wsim[i],
                               coef_t[i].reshape(1, 3),
                               coef_q[i].reshape(1, 3))
        atts.append(att)
    atts.append(_final_att(xt, xq))
    return jnp.stack(atts)[:, :, :NT]


# speedup vs baseline: 8.4412x; 1.2810x over previous
import functools
import math

import jax
import jax.numpy as jnp
from jax import lax
from jax.experimental import pallas as pl
from jax.experimental.pallas import tpu as pltpu
from jax.experimental.pallas import tpu_sc as plsc

NT = 10000
NQ = 256
ET = 320000
EQ = 1024
H = 128
L = 3
NTP = 10240
BT = 2048
NBT = NTP // BT
NC = 2
NS = 16
NW = NC * NS
EPW = ET // NW
KC = 80
NCHUNK = EPW // KC
ROWS_PT = NTP // NS
TPW = NTP // NW
QPW = NQ // NW

_SC_MESH = dict(core_axis_name="c", subcore_axis_name="s")


def _gather_cnt_body(embed_hbm, txp_hbm, qx_hbm, dst2d_hbm, zc_hbm, ones_hbm,
                     et0_hbm, eq0_hbm, cnt_hbm,
                     cnt_sh, idx80, qidx_v, dstc_v, rows_v, ones_v, sem):
    c = lax.axis_index("c")
    s = lax.axis_index("s")
    wid = c * NS + s
    my_rows = pl.multiple_of(s * ROWS_PT, 8)
    pltpu.sync_copy(zc_hbm, cnt_sh.at[pl.ds(my_rows, ROWS_PT)])
    pltpu.sync_copy(ones_hbm, ones_v)

    tb = pl.multiple_of(wid * TPW, 8)

    def tchunk(j, carry):
        off = pl.multiple_of(j * KC, 8)
        pltpu.sync_copy(txp_hbm.at[pl.ds(tb + off, KC)], idx80)
        pltpu.async_copy(embed_hbm.at[idx80], rows_v, sem).wait()
        pltpu.sync_copy(rows_v, et0_hbm.at[pl.ds(tb + off, KC)])
        return carry

    lax.fori_loop(0, TPW // KC, tchunk, 0)

    qb = pl.multiple_of(wid * QPW, 8)
    pltpu.sync_copy(qx_hbm.at[pl.ds(qb, QPW)], qidx_v)
    pltpu.async_copy(embed_hbm.at[qidx_v], rows_v.at[pl.ds(0, QPW)], sem).wait()
    pltpu.sync_copy(rows_v.at[pl.ds(0, QPW)], eq0_hbm.at[pl.ds(qb, QPW)])

    pltpu.sync_copy(dst2d_hbm.at[wid], dstc_v)
    plsc.subcore_barrier()

    def cchunk(j, carry):
        pltpu.sync_copy(ones_v, cnt_sh.at[dstc_v.at[j]], add=True)
        return carry

    lax.fori_loop(0, NCHUNK, cchunk, 0)
    plsc.subcore_barrier()
    pltpu.sync_copy(cnt_sh.at[pl.ds(my_rows, ROWS_PT)],
                    cnt_hbm.at[pl.ds(c * NTP + my_rows, ROWS_PT)])


def _gather_cnt(embed, txp, qx, dst2d, zc, ones_c):
    fn = pl.kernel(
        _gather_cnt_body,
        out_type=[
            jax.ShapeDtypeStruct((NTP, H), jnp.float32),
            jax.ShapeDtypeStruct((NQ, H), jnp.float32),
            jax.ShapeDtypeStruct((NC * NTP, H), jnp.float32),
        ],
        mesh=plsc.VectorSubcoreMesh(**_SC_MESH),
        scratch_types=[
            pltpu.VMEM_SHARED((NTP, H), jnp.float32),
            pltpu.VMEM((KC,), jnp.int32),
            pltpu.VMEM((QPW,), jnp.int32),
            pltpu.VMEM((NCHUNK, KC), jnp.int32),
            pltpu.VMEM((KC, H), jnp.float32),
            pltpu.VMEM((KC, H), jnp.float32),
            pltpu.SemaphoreType.DMA,
        ],
    )
    return fn(embed, txp, qx, dst2d, zc, ones_c)


NBUF = 4


def _seg_sum_body(x_hbm, sd_hbm, za_hbm, out_hbm,
                  agg_sh, i0, i1, i2, i3, r0, r1, r2, r3,
                  si0, si1, si2, si3, sg0, sg1, sg2, sg3,
                  ss0, ss1, ss2, ss3):
    c = lax.axis_index("c")
    s = lax.axis_index("s")
    wid = c * NS + s
    my_rows = pl.multiple_of(s * ROWS_PT, 8)
    pltpu.sync_copy(za_hbm, agg_sh.at[pl.ds(my_rows, ROWS_PT)])

    idx = (i0, i1, i2, i3)
    rows = (r0, r1, r2, r3)
    sis = (si0, si1, si2, si3)
    sgs = (sg0, sg1, sg2, sg3)
    sss = (ss0, ss1, ss2, ss3)

    def i_desc(chk, b):
        return pltpu.make_async_copy(sd_hbm.at[wid, chk], idx[b], sis[b])

    def g_desc(b):
        return pltpu.make_async_copy(x_hbm.at[idx[b].at[0]], rows[b], sgs[b])

    def s_desc(b):
        return pltpu.make_async_copy(rows[b], agg_sh.at[idx[b].at[1]], sss[b])

    plsc.subcore_barrier()

    def grp(g, carry):
        ibase = g * NBUF
        for u in range(NBUF):
            i = ibase + u

            @pl.when(i < NCHUNK)
            def _():
                @pl.when(i >= NBUF)
                def _():
                    s_desc(u).wait()

                i_desc(i, u).start()

            gch = i - 1
            bg = (u + NBUF - 1) % NBUF

            @pl.when(jnp.logical_and(gch >= 0, gch < NCHUNK))
            def _():
                i_desc(gch, bg).wait()
                g_desc(bg).start()

            sch = i - 2
            bs = (u + NBUF - 2) % NBUF

            @pl.when(jnp.logical_and(sch >= 0, sch < NCHUNK))
            def _():
                g_desc(bs).wait()
                s_desc(bs).start(add=True)
        return carry

    lax.fori_loop(0, (NCHUNK + 2 + NBUF - 1) // NBUF, grp, 0)
    for chk in range(NCHUNK - NBUF, NCHUNK):
        s_desc(chk % NBUF).wait()
    plsc.subcore_barrier()
    pltpu.sync_copy(agg_sh.at[pl.ds(my_rows, ROWS_PT)],
                    out_hbm.at[pl.ds(c * NTP + my_rows, ROWS_PT)])


def _seg_sum(x, sd, za):
    fn = pl.kernel(
        _seg_sum_body,
        out_type=jax.ShapeDtypeStruct((NC * NTP, H), jnp.float32),
        mesh=plsc.VectorSubcoreMesh(**_SC_MESH),
        scratch_types=[pltpu.VMEM_SHARED((NTP, H), jnp.float32)]
        + [pltpu.VMEM((2, KC), jnp.int32)] * NBUF
        + [pltpu.VMEM((KC, H), jnp.float32)] * NBUF
        + [pltpu.SemaphoreType.DMA] * (3 * NBUF),
    )
    return fn(x, sd, za)


def _prep_body(qei_ref, mq_ref):
    src = qei_ref[0:1, :]
    dst = qei_ref[1:2, :]
    ids = lax.broadcasted_iota(jnp.int32, (NQ, EQ), 0)
    ohd = (ids == dst).astype(jnp.float32)
    ohs = (ids == src).astype(jnp.float32)
    aq = lax.dot_general(ohd, ohs, (((1,), (1,)), ((), ())),
                         preferred_element_type=jnp.float32)
    cnt = jnp.sum(aq, axis=1, keepdims=True)
    mq_ref[...] = aq / jnp.maximum(cnt, 1.0)


def _prep(qei):
    return pl.pallas_call(
        _prep_body,
        out_shape=jax.ShapeDtypeStruct((NQ, NQ), jnp.float32),
    )(qei)


def _elu(x):
    return jnp.where(x > 0, x, jnp.exp(jnp.minimum(x, 0.0)) - 1.0)


def _mmt(a, b):
    return lax.dot_general(a, b, (((1,), (1,)), ((), ())),
                           preferred_element_type=jnp.float32)


def _sage_body(xt_ref, agg_ref, cnt_ref, wl_ref, bl_ref, wr_ref,
               xq_ref, mq_ref, ht_ref, hq_ref):
    j = pl.program_id(0)
    agg = agg_ref[0] + agg_ref[1]
    cnt = cnt_ref[0, :, 0:1] + cnt_ref[1, :, 0:1]
    mean = agg / jnp.maximum(cnt, 1.0)
    h = _mmt(mean, wl_ref[...]) + bl_ref[...] + _mmt(xt_ref[...], wr_ref[...])
    ht_ref[...] = _elu(h)

    @pl.when(j == 0)
    def _():
        meanq = jnp.dot(mq_ref[...], xq_ref[...],
                        preferred_element_type=jnp.float32)
        hq = _mmt(meanq, wl_ref[...]) + bl_ref[...] + _mmt(xq_ref[...],
                                                           wr_ref[...])
        hq_ref[...] = _elu(hq)


def _sage(xt, agg2, cnt2, wl, bl, wr, xq, mq):
    return pl.pallas_call(
        _sage_body,
        grid=(NBT,),
        in_specs=[
            pl.BlockSpec((BT, H), lambda j: (j, 0)),
            pl.BlockSpec((NC, BT, H), lambda j: (0, j, 0)),
            pl.BlockSpec((NC, BT, H), lambda j: (0, j, 0)),
            pl.BlockSpec((H, H), lambda j: (0, 0)),
            pl.BlockSpec((1, H), lambda j: (0, 0)),
            pl.BlockSpec((H, H), lambda j: (0, 0)),
            pl.BlockSpec((NQ, H), lambda j: (0, 0)),
            pl.BlockSpec((NQ, NQ), lambda j: (0, 0)),
        ],
        out_specs=[
            pl.BlockSpec((BT, H), lambda j: (j, 0)),
            pl.BlockSpec((NQ, H), lambda j: (0, 0)),
        ],
        out_shape=[
            jax.ShapeDtypeStruct((NTP, H), jnp.float32),
            jax.ShapeDtypeStruct((NQ, H), jnp.float32),
        ],
    )(xt, agg2, cnt2, wl, bl, wr, xq, mq)


def _att_mix_body(ht_ref, hq_ref, et0_ref, eq0_ref, wsim_ref, ct_ref, cq_ref,
                  att_ref, xt_ref, xq_ref):
    ht = ht_ref[...]
    hq = hq_ref[...]
    sq = jnp.dot(hq, wsim_ref[...], preferred_element_type=jnp.float32)
    logits = _mmt(sq, ht) * (1.0 / math.sqrt(H))
    col = lax.broadcasted_iota(jnp.int32, (NQ, NTP), 1)
    logits = jnp.where(col < NT, logits, -1e9)
    m = jnp.max(logits, axis=1, keepdims=True)
    e = jnp.exp(logits - m)
    att = e / jnp.sum(e, axis=1, keepdims=True)
    att_ref[...] = att
    cross_q = jnp.dot(att, ht, preferred_element_type=jnp.float32)
    cross_t = lax.dot_general(att, hq, (((0,), (0,)), ((), ())),
                              preferred_element_type=jnp.float32)
    ct = ct_ref[...]
    cq = cq_ref[...]
    xq_ref[...] = (cq[0:1, 0:1] * eq0_ref[...] + cq[0:1, 1:2] * hq
                   + cq[0:1, 2:3] * cross_q)
    xt_ref[...] = (ct[0:1, 0:1] * et0_ref[...] + ct[0:1, 1:2] * ht
                   + ct[0:1, 2:3] * cross_t)


def _att_mix(ht, hq, et0, eq0, wsim, ct, cq):
    return pl.pallas_call(
        _att_mix_body,
        out_shape=[
            jax.ShapeDtypeStruct((NQ, NTP), jnp.float32),
            jax.ShapeDtypeStruct((NTP, H), jnp.float32),
            jax.ShapeDtypeStruct((NQ, H), jnp.float32),
        ],
    )(ht, hq, et0, eq0, wsim, ct, cq)


def _final_att_body(xt_ref, xq_ref, att_ref):
    logits = _mmt(xq_ref[...], xt_ref[...]) * (1.0 / math.sqrt(H))
    col = lax.broadcasted_iota(jnp.int32, (NQ, NTP), 1)
    logits = jnp.where(col < NT, logits, -1e9)
    m = jnp.max(logits, axis=1, keepdims=True)
    e = jnp.exp(logits - m)
    att_ref[...] = e / jnp.sum(e, axis=1, keepdims=True)


def _final_att(xt, xq):
    return pl.pallas_call(
        _final_att_body,
        out_shape=jax.ShapeDtypeStruct((NQ, NTP), jnp.float32),
    )(xt, xq)


def kernel(target_x, query_x, target_edge_index, query_edge_index, mask,
           embed, Wl, bl, Wr, Wsim, coef_t, coef_q):
    del mask
    tx = target_x.astype(jnp.int32)
    qx = query_x.astype(jnp.int32)
    txp = jnp.pad(tx, (0, NTP - NT))
    src = target_edge_index[0].astype(jnp.int32)
    dst = target_edge_index[1].astype(jnp.int32)
    dst2d = dst.reshape(NW, NCHUNK, KC)
    sd = jnp.stack([src.reshape(NW, NCHUNK, KC), dst2d], axis=2)
    za = jnp.zeros((ROWS_PT, H), jnp.float32)
    ones_c = jnp.ones((KC, H), jnp.float32)

    embed_p = jnp.pad(embed, ((0, -embed.shape[0] % 8), (0, 0)))
    et0, eq0, cnt2f = _gather_cnt(embed_p, txp, qx, dst2d, za, ones_c)
    cnt2 = cnt2f.reshape(NC, NTP, H)
    mq = _prep(query_edge_index.astype(jnp.int32))

    xt, xq = et0, eq0
    atts = []
    for i in range(L):
        agg2 = _seg_sum(xt, sd, za).reshape(NC, NTP, H)
        ht, hq = _sage(xt, agg2, cnt2, Wl[i], bl[i].reshape(1, H), Wr[i],
                       xq, mq)
        att, xt, xq = _att_mix(ht, hq, et0, eq0, Wsim[i],
                               coef_t[i].reshape(1, 3),
                               coef_q[i].reshape(1, 3))
        atts.append(att)
    atts.append(_final_att(xt, xq))
    return jnp.stack(atts)[:, :, :NT]


# speedup vs baseline: 8.4599x; 1.0022x over previous
import functools
import math

import jax
import jax.numpy as jnp
from jax import lax
from jax.experimental import pallas as pl
from jax.experimental.pallas import tpu as pltpu
from jax.experimental.pallas import tpu_sc as plsc

NT = 10000
NQ = 256
ET = 320000
EQ = 1024
H = 128
L = 3
NTP = 10240
BT = 2048
NBT = NTP // BT
NC = 2
NS = 16
NW = NC * NS
EPW = ET // NW
KC = 80
NCHUNK = EPW // KC
ROWS_PT = NTP // NS
TPW = NTP // NW
QPW = NQ // NW

_SC_MESH = dict(core_axis_name="c", subcore_axis_name="s")


def _gather_cnt_body(embed_hbm, txp_hbm, qx_hbm, dst2d_hbm, zc_hbm, ones_hbm,
                     et0_hbm, eq0_hbm, cnt_hbm,
                     cnt_sh, idx80, qidx_v, dstc_v, rows_v, ones_v, sem,
                     sc0, sc1, sc2, sc3):
    c = lax.axis_index("c")
    s = lax.axis_index("s")
    wid = c * NS + s
    scs = (sc0, sc1, sc2, sc3)
    my_rows = pl.multiple_of(s * ROWS_PT, 8)
    pltpu.sync_copy(zc_hbm, cnt_sh.at[pl.ds(my_rows, ROWS_PT)])
    pltpu.sync_copy(ones_hbm, ones_v)

    tb = pl.multiple_of(wid * TPW, 8)

    def tchunk(j, carry):
        off = pl.multiple_of(j * KC, 8)
        pltpu.sync_copy(txp_hbm.at[pl.ds(tb + off, KC)], idx80)
        pltpu.async_copy(embed_hbm.at[idx80], rows_v, sem).wait()
        pltpu.sync_copy(rows_v, et0_hbm.at[pl.ds(tb + off, KC)])
        return carry

    lax.fori_loop(0, TPW // KC, tchunk, 0)

    qb = pl.multiple_of(wid * QPW, 8)
    pltpu.sync_copy(qx_hbm.at[pl.ds(qb, QPW)], qidx_v)
    pltpu.async_copy(embed_hbm.at[qidx_v], rows_v.at[pl.ds(0, QPW)], sem).wait()
    pltpu.sync_copy(rows_v.at[pl.ds(0, QPW)], eq0_hbm.at[pl.ds(qb, QPW)])

    pltpu.sync_copy(dst2d_hbm.at[wid], dstc_v)
    plsc.subcore_barrier()

    def c_desc(chk, b):
        return pltpu.make_async_copy(ones_v, cnt_sh.at[dstc_v.at[chk]],
                                     scs[b])

    def cgrp(g, carry):
        ibase = g * NBUF
        for u in range(NBUF):
            chk = ibase + u

            @pl.when(chk < NCHUNK)
            def _():
                @pl.when(chk >= NBUF)
                def _():
                    c_desc(chk - NBUF, u).wait()

                c_desc(chk, u).start(add=True)
        return carry

    lax.fori_loop(0, (NCHUNK + NBUF - 1) // NBUF, cgrp, 0)
    for chk in range(NCHUNK - NBUF, NCHUNK):
        c_desc(chk, chk % NBUF).wait()
    plsc.subcore_barrier()
    pltpu.sync_copy(cnt_sh.at[pl.ds(my_rows, ROWS_PT)],
                    cnt_hbm.at[pl.ds(c * NTP + my_rows, ROWS_PT)])


def _gather_cnt(embed, txp, qx, dst2d, zc, ones_c):
    fn = pl.kernel(
        _gather_cnt_body,
        out_type=[
            jax.ShapeDtypeStruct((NTP, H), jnp.float32),
            jax.ShapeDtypeStruct((NQ, H), jnp.float32),
            jax.ShapeDtypeStruct((NC * NTP, H), jnp.float32),
        ],
        mesh=plsc.VectorSubcoreMesh(**_SC_MESH),
        scratch_types=[
            pltpu.VMEM_SHARED((NTP, H), jnp.float32),
            pltpu.VMEM((KC,), jnp.int32),
            pltpu.VMEM((QPW,), jnp.int32),
            pltpu.VMEM((NCHUNK, KC), jnp.int32),
            pltpu.VMEM((KC, H), jnp.float32),
            pltpu.VMEM((KC, H), jnp.float32),
        ] + [pltpu.SemaphoreType.DMA] * 5,
    )
    return fn(embed, txp, qx, dst2d, zc, ones_c)


NBUF = 4


def _seg_sum_body(x_hbm, sd_hbm, za_hbm, out_hbm,
                  agg_sh, i0, i1, i2, i3, r0, r1, r2, r3,
                  si0, si1, si2, si3, sg0, sg1, sg2, sg3,
                  ss0, ss1, ss2, ss3):
    c = lax.axis_index("c")
    s = lax.axis_index("s")
    wid = c * NS + s
    my_rows = pl.multiple_of(s * ROWS_PT, 8)
    pltpu.sync_copy(za_hbm, agg_sh.at[pl.ds(my_rows, ROWS_PT)])

    idx = (i0, i1, i2, i3)
    rows = (r0, r1, r2, r3)
    sis = (si0, si1, si2, si3)
    sgs = (sg0, sg1, sg2, sg3)
    sss = (ss0, ss1, ss2, ss3)

    def i_desc(chk, b):
        return pltpu.make_async_copy(sd_hbm.at[wid, chk], idx[b], sis[b])

    def g_desc(b):
        return pltpu.make_async_copy(x_hbm.at[idx[b].at[0]], rows[b], sgs[b])

    def s_desc(b):
        return pltpu.make_async_copy(rows[b], agg_sh.at[idx[b].at[1]], sss[b])

    plsc.subcore_barrier()

    def grp(g, carry):
        ibase = g * NBUF
        for u in range(NBUF):
            i = ibase + u

            @pl.when(i < NCHUNK)
            def _():
                @pl.when(i >= NBUF)
                def _():
                    s_desc(u).wait()

                i_desc(i, u).start()

            gch = i - 1
            bg = (u + NBUF - 1) % NBUF

            @pl.when(jnp.logical_and(gch >= 0, gch < NCHUNK))
            def _():
                i_desc(gch, bg).wait()
                g_desc(bg).start()

            sch = i - 2
            bs = (u + NBUF - 2) % NBUF

            @pl.when(jnp.logical_and(sch >= 0, sch < NCHUNK))
            def _():
                g_desc(bs).wait()
                s_desc(bs).start(add=True)
        return carry

    lax.fori_loop(0, (NCHUNK + 2 + NBUF - 1) // NBUF, grp, 0)
    for chk in range(NCHUNK - NBUF, NCHUNK):
        s_desc(chk % NBUF).wait()
    plsc.subcore_barrier()
    pltpu.sync_copy(agg_sh.at[pl.ds(my_rows, ROWS_PT)],
                    out_hbm.at[pl.ds(c * NTP + my_rows, ROWS_PT)])


def _seg_sum(x, sd, za):
    fn = pl.kernel(
        _seg_sum_body,
        out_type=jax.ShapeDtypeStruct((NC * NTP, H), jnp.float32),
        mesh=plsc.VectorSubcoreMesh(**_SC_MESH),
        scratch_types=[pltpu.VMEM_SHARED((NTP, H), jnp.float32)]
        + [pltpu.VMEM((2, KC), jnp.int32)] * NBUF
        + [pltpu.VMEM((KC, H), jnp.float32)] * NBUF
        + [pltpu.SemaphoreType.DMA] * (3 * NBUF),
    )
    return fn(x, sd, za)


def _prep_body(qei_ref, mq_ref):
    src = qei_ref[0:1, :]
    dst = qei_ref[1:2, :]
    ids = lax.broadcasted_iota(jnp.int32, (NQ, EQ), 0)
    ohd = (ids == dst).astype(jnp.float32)
    ohs = (ids == src).astype(jnp.float32)
    aq = lax.dot_general(ohd, ohs, (((1,), (1,)), ((), ())),
                         preferred_element_type=jnp.float32)
    cnt = jnp.sum(aq, axis=1, keepdims=True)
    mq_ref[...] = aq / jnp.maximum(cnt, 1.0)


def _prep(qei):
    return pl.pallas_call(
        _prep_body,
        out_shape=jax.ShapeDtypeStruct((NQ, NQ), jnp.float32),
    )(qei)


def _elu(x):
    return jnp.where(x > 0, x, jnp.exp(jnp.minimum(x, 0.0)) - 1.0)


def _mmt(a, b):
    return lax.dot_general(a, b, (((1,), (1,)), ((), ())),
                           preferred_element_type=jnp.float32)


def _sage_body(xt_ref, agg_ref, cnt_ref, wl_ref, bl_ref, wr_ref,
               xq_ref, mq_ref, ht_ref, hq_ref):
    j = pl.program_id(0)
    agg = agg_ref[0] + agg_ref[1]
    cnt = cnt_ref[0, :, 0:1] + cnt_ref[1, :, 0:1]
    mean = agg / jnp.maximum(cnt, 1.0)
    h = _mmt(mean, wl_ref[...]) + bl_ref[...] + _mmt(xt_ref[...], wr_ref[...])
    ht_ref[...] = _elu(h)

    @pl.when(j == 0)
    def _():
        meanq = jnp.dot(mq_ref[...], xq_ref[...],
                        preferred_element_type=jnp.float32)
        hq = _mmt(meanq, wl_ref[...]) + bl_ref[...] + _mmt(xq_ref[...],
                                                           wr_ref[...])
        hq_ref[...] = _elu(hq)


def _sage(xt, agg2, cnt2, wl, bl, wr, xq, mq):
    return pl.pallas_call(
        _sage_body,
        grid=(NBT,),
        in_specs=[
            pl.BlockSpec((BT, H), lambda j: (j, 0)),
            pl.BlockSpec((NC, BT, H), lambda j: (0, j, 0)),
            pl.BlockSpec((NC, BT, H), lambda j: (0, j, 0)),
            pl.BlockSpec((H, H), lambda j: (0, 0)),
            pl.BlockSpec((1, H), lambda j: (0, 0)),
            pl.BlockSpec((H, H), lambda j: (0, 0)),
            pl.BlockSpec((NQ, H), lambda j: (0, 0)),
            pl.BlockSpec((NQ, NQ), lambda j: (0, 0)),
        ],
        out_specs=[
            pl.BlockSpec((BT, H), lambda j: (j, 0)),
            pl.BlockSpec((NQ, H), lambda j: (0, 0)),
        ],
        out_shape=[
            jax.ShapeDtypeStruct((NTP, H), jnp.float32),
            jax.ShapeDtypeStruct((NQ, H), jnp.float32),
        ],
    )(xt, agg2, cnt2, wl, bl, wr, xq, mq)


def _att_mix_body(ht_ref, hq_ref, et0_ref, eq0_ref, wsim_ref, ct_ref, cq_ref,
                  att_ref, xt_ref, xq_ref):
    ht = ht_ref[...]
    hq = hq_ref[...]
    sq = jnp.dot(hq, wsim_ref[...], preferred_element_type=jnp.float32)
    logits = _mmt(sq, ht) * (1.0 / math.sqrt(H))
    col = lax.broadcasted_iota(jnp.int32, (NQ, NTP), 1)
    logits = jnp.where(col < NT, logits, -1e9)
    m = jnp.max(logits, axis=1, keepdims=True)
    e = jnp.exp(logits - m)
    att = e / jnp.sum(e, axis=1, keepdims=True)
    att_ref[...] = att
    cross_q = jnp.dot(att, ht, preferred_element_type=jnp.float32)
    cross_t = lax.dot_general(att, hq, (((0,), (0,)), ((), ())),
                              preferred_element_type=jnp.float32)
    ct = ct_ref[...]
    cq = cq_ref[...]
    xq_ref[...] = (cq[0:1, 0:1] * eq0_ref[...] + cq[0:1, 1:2] * hq
                   + cq[0:1, 2:3] * cross_q)
    xt_ref[...] = (ct[0:1, 0:1] * et0_ref[...] + ct[0:1, 1:2] * ht
                   + ct[0:1, 2:3] * cross_t)


def _att_mix(ht, hq, et0, eq0, wsim, ct, cq):
    return pl.pallas_call(
        _att_mix_body,
        out_shape=[
            jax.ShapeDtypeStruct((NQ, NTP), jnp.float32),
            jax.ShapeDtypeStruct((NTP, H), jnp.float32),
            jax.ShapeDtypeStruct((NQ, H), jnp.float32),
        ],
    )(ht, hq, et0, eq0, wsim, ct, cq)


def _final_att_body(xt_ref, xq_ref, att_ref):
    logits = _mmt(xq_ref[...], xt_ref[...]) * (1.0 / math.sqrt(H))
    col = lax.broadcasted_iota(jnp.int32, (NQ, NTP), 1)
    logits = jnp.where(col < NT, logits, -1e9)
    m = jnp.max(logits, axis=1, keepdims=True)
    e = jnp.exp(logits - m)
    att_ref[...] = e / jnp.sum(e, axis=1, keepdims=True)


def _final_att(xt, xq):
    return pl.pallas_call(
        _final_att_body,
        out_shape=jax.ShapeDtypeStruct((NQ, NTP), jnp.float32),
    )(xt, xq)


def kernel(target_x, query_x, target_edge_index, query_edge_index, mask,
           embed, Wl, bl, Wr, Wsim, coef_t, coef_q):
    del mask
    tx = target_x.astype(jnp.int32)
    qx = query_x.astype(jnp.int32)
    txp = jnp.pad(tx, (0, NTP - NT))
    src = target_edge_index[0].astype(jnp.int32)
    dst = target_edge_index[1].astype(jnp.int32)
    dst2d = dst.reshape(NW, NCHUNK, KC)
    sd = jnp.stack([src.reshape(NW, NCHUNK, KC), dst2d], axis=2)
    za = jnp.zeros((ROWS_PT, H), jnp.float32)
    ones_c = jnp.ones((KC, H), jnp.float32)

    embed_p = jnp.pad(embed, ((0, -embed.shape[0] % 8), (0, 0)))
    et0, eq0, cnt2f = _gather_cnt(embed_p, txp, qx, dst2d, za, ones_c)
    cnt2 = cnt2f.reshape(NC, NTP, H)
    mq = _prep(query_edge_index.astype(jnp.int32))

    xt, xq = et0, eq0
    atts = []
    for i in range(L):
        agg2 = _seg_sum(xt, sd, za).reshape(NC, NTP, H)
        ht, hq = _sage(xt, agg2, cnt2, Wl[i], bl[i].reshape(1, H), Wr[i],
                       xq, mq)
        att, xt, xq = _att_mix(ht, hq, et0, eq0, Wsim[i],
                               coef_t[i].reshape(1, 3),
                               coef_q[i].reshape(1, 3))
        atts.append(att)
    atts.append(_final_att(xt, xq))
    return jnp.stack(atts)[:, :, :NT]


# speedup vs baseline: 8.8237x; 1.0430x over previous
import functools
import math

import jax
import jax.numpy as jnp
from jax import lax
from jax.experimental import pallas as pl
from jax.experimental.pallas import tpu as pltpu
from jax.experimental.pallas import tpu_sc as plsc

NT = 10000
NQ = 256
ET = 320000
EQ = 1024
H = 128
L = 3
NTP = 10240
BT = 2048
NBT = NTP // BT
NC = 2
NS = 16
NW = NC * NS
EPW = ET // NW
KC = 80
NCHUNK = EPW // KC
ROWS_PT = NTP // NS
TPW = NTP // NW
QPW = NQ // NW

_SC_MESH = dict(core_axis_name="c", subcore_axis_name="s")


def _gather_cnt_body(embed_hbm, txp_hbm, qx_hbm, dst2d_hbm, zc_hbm, ones_hbm,
                     et0_hbm, eq0_hbm, cnt_hbm,
                     cnt_sh, idx80, qidx_v, dstc_v, rows_v, ones_v, sem,
                     sc0, sc1, sc2, sc3):
    c = lax.axis_index("c")
    s = lax.axis_index("s")
    wid = c * NS + s
    scs = (sc0, sc1, sc2, sc3)
    my_rows = pl.multiple_of(s * ROWS_PT, 8)
    pltpu.sync_copy(zc_hbm, cnt_sh.at[pl.ds(my_rows, ROWS_PT)])
    pltpu.sync_copy(ones_hbm, ones_v)

    tb = pl.multiple_of(wid * TPW, 8)

    def tchunk(j, carry):
        off = pl.multiple_of(j * KC, 8)
        pltpu.sync_copy(txp_hbm.at[pl.ds(tb + off, KC)], idx80)
        pltpu.async_copy(embed_hbm.at[idx80], rows_v, sem).wait()
        pltpu.sync_copy(rows_v, et0_hbm.at[pl.ds(tb + off, KC)])
        return carry

    lax.fori_loop(0, TPW // KC, tchunk, 0)

    qb = pl.multiple_of(wid * QPW, 8)
    pltpu.sync_copy(qx_hbm.at[pl.ds(qb, QPW)], qidx_v)
    pltpu.async_copy(embed_hbm.at[qidx_v], rows_v.at[pl.ds(0, QPW)], sem).wait()
    pltpu.sync_copy(rows_v.at[pl.ds(0, QPW)], eq0_hbm.at[pl.ds(qb, QPW)])

    pltpu.sync_copy(dst2d_hbm.at[wid], dstc_v)
    plsc.subcore_barrier()

    def c_desc(chk, b):
        return pltpu.make_async_copy(ones_v, cnt_sh.at[dstc_v.at[chk]],
                                     scs[b])

    def cgrp(g, carry):
        ibase = g * NBUF
        for u in range(NBUF):
            chk = ibase + u

            @pl.when(chk < NCHUNK)
            def _():
                @pl.when(chk >= NBUF)
                def _():
                    c_desc(chk - NBUF, u).wait()

                c_desc(chk, u).start(add=True)
        return carry

    lax.fori_loop(0, (NCHUNK + NBUF - 1) // NBUF, cgrp, 0)
    for chk in range(NCHUNK - NBUF, NCHUNK):
        c_desc(chk, chk % NBUF).wait()
    plsc.subcore_barrier()
    pltpu.sync_copy(cnt_sh.at[pl.ds(my_rows, ROWS_PT)],
                    cnt_hbm.at[pl.ds(c * NTP + my_rows, ROWS_PT)])


def _gather_cnt(embed, txp, qx, dst2d, zc, ones_c):
    fn = pl.kernel(
        _gather_cnt_body,
        out_type=[
            jax.ShapeDtypeStruct((NTP, H), jnp.float32),
            jax.ShapeDtypeStruct((NQ, H), jnp.float32),
            jax.ShapeDtypeStruct((NC * NTP, H), jnp.float32),
        ],
        mesh=plsc.VectorSubcoreMesh(**_SC_MESH),
        scratch_types=[
            pltpu.VMEM_SHARED((NTP, H), jnp.float32),
            pltpu.VMEM((KC,), jnp.int32),
            pltpu.VMEM((QPW,), jnp.int32),
            pltpu.VMEM((NCHUNK, KC), jnp.int32),
            pltpu.VMEM((KC, H), jnp.float32),
            pltpu.VMEM((KC, H), jnp.float32),
        ] + [pltpu.SemaphoreType.DMA] * 5,
    )
    return fn(embed, txp, qx, dst2d, zc, ones_c)


NBUF = 4


def _seg_sum_body(x_hbm, sd_hbm, za_hbm, out_hbm,
                  agg_sh, i0, i1, i2, i3, r0, r1, r2, r3,
                  si0, si1, si2, si3, sg0, sg1, sg2, sg3,
                  ss0, ss1, ss2, ss3):
    c = lax.axis_index("c")
    s = lax.axis_index("s")
    wid = c * NS + s
    my_rows = pl.multiple_of(s * ROWS_PT, 8)
    pltpu.sync_copy(za_hbm, agg_sh.at[pl.ds(my_rows, ROWS_PT)])

    idx = (i0, i1, i2, i3)
    rows = (r0, r1, r2, r3)
    sis = (si0, si1, si2, si3)
    sgs = (sg0, sg1, sg2, sg3)
    sss = (ss0, ss1, ss2, ss3)

    def i_desc(chk, b):
        return pltpu.make_async_copy(sd_hbm.at[wid, chk], idx[b], sis[b])

    def g_desc(b):
        return pltpu.make_async_copy(x_hbm.at[idx[b].at[0]], rows[b], sgs[b])

    def s_desc(b):
        return pltpu.make_async_copy(rows[b], agg_sh.at[idx[b].at[1]], sss[b])

    plsc.subcore_barrier()

    def grp(g, carry):
        ibase = g * NBUF
        for u in range(NBUF):
            i = ibase + u

            @pl.when(i < NCHUNK)
            def _():
                @pl.when(i >= NBUF)
                def _():
                    s_desc(u).wait()

                i_desc(i, u).start()

            gch = i - 1
            bg = (u + NBUF - 1) % NBUF

            @pl.when(jnp.logical_and(gch >= 0, gch < NCHUNK))
            def _():
                i_desc(gch, bg).wait()
                g_desc(bg).start()

            sch = i - 2
            bs = (u + NBUF - 2) % NBUF

            @pl.when(jnp.logical_and(sch >= 0, sch < NCHUNK))
            def _():
                g_desc(bs).wait()
                s_desc(bs).start(add=True)
        return carry

    lax.fori_loop(0, (NCHUNK + 2 + NBUF - 1) // NBUF, grp, 0)
    for chk in range(NCHUNK - NBUF, NCHUNK):
        s_desc(chk % NBUF).wait()
    plsc.subcore_barrier()
    pltpu.sync_copy(agg_sh.at[pl.ds(my_rows, ROWS_PT)],
                    out_hbm.at[pl.ds(c * NTP + my_rows, ROWS_PT)])


def _seg_sum(x, sd, za):
    fn = pl.kernel(
        _seg_sum_body,
        out_type=jax.ShapeDtypeStruct((NC * NTP, H), jnp.float32),
        mesh=plsc.VectorSubcoreMesh(**_SC_MESH),
        scratch_types=[pltpu.VMEM_SHARED((NTP, H), jnp.float32)]
        + [pltpu.VMEM((2, KC), jnp.int32)] * NBUF
        + [pltpu.VMEM((KC, H), jnp.float32)] * NBUF
        + [pltpu.SemaphoreType.DMA] * (3 * NBUF),
    )
    return fn(x, sd, za)


def _prep_body(qei_ref, mq_ref):
    src = qei_ref[0:1, :]
    dst = qei_ref[1:2, :]
    ids = lax.broadcasted_iota(jnp.int32, (NQ, EQ), 0)
    ohd = (ids == dst).astype(jnp.float32)
    ohs = (ids == src).astype(jnp.float32)
    aq = lax.dot_general(ohd, ohs, (((1,), (1,)), ((), ())),
                         preferred_element_type=jnp.float32)
    cnt = jnp.sum(aq, axis=1, keepdims=True)
    mq_ref[...] = aq / jnp.maximum(cnt, 1.0)


def _prep(qei):
    return pl.pallas_call(
        _prep_body,
        out_shape=jax.ShapeDtypeStruct((NQ, NQ), jnp.float32),
    )(qei)


def _elu(x):
    return jnp.where(x > 0, x, jnp.exp(jnp.minimum(x, 0.0)) - 1.0)


def _mmt(a, b):
    return lax.dot_general(a, b, (((1,), (1,)), ((), ())),
                           preferred_element_type=jnp.float32)


def _sage_body(xt_ref, agg_ref, cnt_ref, wl_ref, bl_ref, wr_ref,
               xq_ref, mq_ref, ht_ref, hq_ref):
    j = pl.program_id(0)
    agg = agg_ref[0] + agg_ref[1]
    cnt = cnt_ref[0, :, 0:1] + cnt_ref[1, :, 0:1]
    mean = agg / jnp.maximum(cnt, 1.0)
    h = _mmt(mean, wl_ref[...]) + bl_ref[...] + _mmt(xt_ref[...], wr_ref[...])
    ht_ref[...] = _elu(h)

    @pl.when(j == 0)
    def _():
        meanq = jnp.dot(mq_ref[...], xq_ref[...],
                        preferred_element_type=jnp.float32)
        hq = _mmt(meanq, wl_ref[...]) + bl_ref[...] + _mmt(xq_ref[...],
                                                           wr_ref[...])
        hq_ref[...] = _elu(hq)


def _sage(xt, agg2, cnt2, wl, bl, wr, xq, mq):
    return pl.pallas_call(
        _sage_body,
        grid=(NBT,),
        in_specs=[
            pl.BlockSpec((BT, H), lambda j: (j, 0)),
            pl.BlockSpec((NC, BT, H), lambda j: (0, j, 0)),
            pl.BlockSpec((NC, BT, H), lambda j: (0, j, 0)),
            pl.BlockSpec((H, H), lambda j: (0, 0)),
            pl.BlockSpec((1, H), lambda j: (0, 0)),
            pl.BlockSpec((H, H), lambda j: (0, 0)),
            pl.BlockSpec((NQ, H), lambda j: (0, 0)),
            pl.BlockSpec((NQ, NQ), lambda j: (0, 0)),
        ],
        out_specs=[
            pl.BlockSpec((BT, H), lambda j: (j, 0)),
            pl.BlockSpec((NQ, H), lambda j: (0, 0)),
        ],
        out_shape=[
            jax.ShapeDtypeStruct((NTP, H), jnp.float32),
            jax.ShapeDtypeStruct((NQ, H), jnp.float32),
        ],
    )(xt, agg2, cnt2, wl, bl, wr, xq, mq)


def _softmax_row(logits):
    col = lax.broadcasted_iota(jnp.int32, (NQ, NTP), 1)
    logits = jnp.where(col < NT, logits, -1e9)
    m = jnp.max(logits, axis=1, keepdims=True)
    e = jnp.exp(logits - m)
    return e / jnp.sum(e, axis=1, keepdims=True)


def _att_mix_body(ht_ref, hq_ref, et0_ref, eq0_ref, wsim_ref, ct_ref, cq_ref,
                  abuf_ref, att_ref, xt_ref, xq_ref):
    del abuf_ref
    ht = ht_ref[...]
    hq = hq_ref[...]
    sq = jnp.dot(hq, wsim_ref[...], preferred_element_type=jnp.float32)
    att = _softmax_row(_mmt(sq, ht) * (1.0 / math.sqrt(H)))
    att_ref[0] = att[:, :NT]
    cross_q = jnp.dot(att, ht, preferred_element_type=jnp.float32)
    cross_t = lax.dot_general(att, hq, (((0,), (0,)), ((), ())),
                              preferred_element_type=jnp.float32)
    ct = ct_ref[...]
    cq = cq_ref[...]
    xq_ref[...] = (cq[0:1, 0:1] * eq0_ref[...] + cq[0:1, 1:2] * hq
                   + cq[0:1, 2:3] * cross_q)
    xt_ref[...] = (ct[0:1, 0:1] * et0_ref[...] + ct[0:1, 1:2] * ht
                   + ct[0:1, 2:3] * cross_t)


def _full(shape):
    return pl.BlockSpec(shape, lambda j: tuple(0 for _ in shape))


def _att_mix(i, ht, hq, et0, eq0, wsim, ct, cq, abuf):
    dense = [_full((NTP, H)), _full((NQ, H)), _full((NTP, H)), _full((NQ, H)),
             _full((H, H)), _full((1, 3)), _full((1, 3))]
    args = (ht, hq, et0, eq0, wsim, ct, cq)
    if abuf is None:
        def body(ht_r, hq_r, et0_r, eq0_r, w_r, ct_r, cq_r, att_r, xt_r, xq_r):
            _att_mix_body(ht_r, hq_r, et0_r, eq0_r, w_r, ct_r, cq_r, None,
                          att_r, xt_r, xq_r)
        in_specs, aliases = dense, {}
    else:
        body = _att_mix_body
        in_specs = dense + [pl.BlockSpec(memory_space=pl.ANY)]
        args = args + (abuf,)
        aliases = {7: 0}
    return pl.pallas_call(
        body,
        grid=(1,),
        in_specs=in_specs,
        out_specs=[
            pl.BlockSpec((1, NQ, NT), lambda j, i=i: (i, 0, 0)),
            _full((NTP, H)),
            _full((NQ, H)),
        ],
        out_shape=[
            jax.ShapeDtypeStruct((L + 1, NQ, NT), jnp.float32),
            jax.ShapeDtypeStruct((NTP, H), jnp.float32),
            jax.ShapeDtypeStruct((NQ, H), jnp.float32),
        ],
        input_output_aliases=aliases,
    )(*args)


def _final_att_body(xt_ref, xq_ref, abuf_ref, att_ref):
    del abuf_ref
    att = _softmax_row(_mmt(xq_ref[...], xt_ref[...]) * (1.0 / math.sqrt(H)))
    att_ref[0] = att[:, :NT]


def _final_att(xt, xq, abuf):
    return pl.pallas_call(
        _final_att_body,
        grid=(1,),
        in_specs=[
            _full((NTP, H)), _full((NQ, H)),
            pl.BlockSpec(memory_space=pl.ANY),
        ],
        out_specs=pl.BlockSpec((1, NQ, NT), lambda j: (L, 0, 0)),
        out_shape=jax.ShapeDtypeStruct((L + 1, NQ, NT), jnp.float32),
        input_output_aliases={2: 0},
    )(xt, xq, abuf)


def kernel(target_x, query_x, target_edge_index, query_edge_index, mask,
           embed, Wl, bl, Wr, Wsim, coef_t, coef_q):
    del mask
    tx = target_x.astype(jnp.int32)
    qx = query_x.astype(jnp.int32)
    txp = jnp.pad(tx, (0, NTP - NT))
    src = target_edge_index[0].astype(jnp.int32)
    dst = target_edge_index[1].astype(jnp.int32)
    dst2d = dst.reshape(NW, NCHUNK, KC)
    sd = jnp.stack([src.reshape(NW, NCHUNK, KC), dst2d], axis=2)
    za = jnp.zeros((ROWS_PT, H), jnp.float32)
    ones_c = jnp.ones((KC, H), jnp.float32)

    embed_p = jnp.pad(embed, ((0, -embed.shape[0] % 8), (0, 0)))
    et0, eq0, cnt2f = _gather_cnt(embed_p, txp, qx, dst2d, za, ones_c)
    cnt2 = cnt2f.reshape(NC, NTP, H)
    mq = _prep(query_edge_index.astype(jnp.int32))

    xt, xq = et0, eq0
    abuf = None
    for i in range(L):
        agg2 = _seg_sum(xt, sd, za).reshape(NC, NTP, H)
        ht, hq = _sage(xt, agg2, cnt2, Wl[i], bl[i].reshape(1, H), Wr[i],
                       xq, mq)
        abuf, xt, xq = _att_mix(i, ht, hq, et0, eq0, Wsim[i],
                                coef_t[i].reshape(1, 3),
                                coef_q[i].reshape(1, 3), abuf)
    return _final_att(xt, xq, abuf)
